# baseline (speedup 1.0000x reference)
import functools

import jax
import jax.numpy as jnp
from jax.experimental import pallas as pl


def _body(a_ref, q_ref, o_ref, *, pblk, f):
    for p in range(pblk):
        a = a_ref[:, p * f:(p + 1) * f]
        q = q_ref[p]
        sim = jax.lax.dot_general(
            a, q, (((1,), (1,)), ((), ())),
            preferred_element_type=jnp.float32)
        idx = jnp.argmax(sim, axis=1)
        m = sim.shape[1]
        onehot = (idx[:, None] == jax.lax.broadcasted_iota(jnp.int32, (1, m), 1)
                  ).astype(jnp.float32)
        o_ref[:, p * f:(p + 1) * f] = jax.lax.dot_general(
            onehot, q, (((1,), (0,)), ((), ())),
            preferred_element_type=jnp.float32)


@jax.jit
def kernel(patch_features, queue):
    b, p_total, f = patch_features.shape
    _, m, _ = queue.shape
    pblk = 8
    grid = (p_total // pblk,)
    a2 = patch_features.reshape(b, p_total * f)
    out = pl.pallas_call(
        functools.partial(_body, pblk=pblk, f=f),
        grid=grid,
        in_specs=[
            pl.BlockSpec((b, pblk * f), lambda i: (0, i)),
            pl.BlockSpec((pblk, m, f), lambda i: (i, 0, 0)),
        ],
        out_specs=pl.BlockSpec((b, pblk * f), lambda i: (0, i)),
        out_shape=jax.ShapeDtypeStruct((b, p_total * f), jnp.float32),
    )(a2, queue)
    return out.reshape(b, p_total, f)


# speedup vs baseline: 2.2937x; 2.2937x over previous
import functools

import jax
import jax.numpy as jnp
from jax.experimental import pallas as pl


def _body(a_ref, q_ref, o_ref, *, pblk):
    m = q_ref.shape[1]
    iota = jax.lax.broadcasted_iota(jnp.int32, (1, m), 1)
    sims = [jax.lax.dot_general(
        a_ref[:, p, :], q_ref[p], (((1,), (1,)), ((), ())),
        preferred_element_type=jnp.float32) for p in range(pblk)]
    onehots = [(jnp.argmax(sim, axis=1)[:, None] == iota).astype(jnp.float32)
               for sim in sims]
    for p in range(pblk):
        o_ref[:, p, :] = jax.lax.dot_general(
            onehots[p], q_ref[p], (((1,), (0,)), ((), ())),
            preferred_element_type=jnp.float32)


@jax.jit
def kernel(patch_features, queue):
    b, p_total, f = patch_features.shape
    _, m, _ = queue.shape
    pblk = 8
    grid = (p_total // pblk,)
    return pl.pallas_call(
        functools.partial(_body, pblk=pblk),
        grid=grid,
        in_specs=[
            pl.BlockSpec((b, pblk, f), lambda i: (0, i, 0)),
            pl.BlockSpec((pblk, m, f), lambda i: (i, 0, 0)),
        ],
        out_specs=pl.BlockSpec((b, pblk, f), lambda i: (0, i, 0)),
        out_shape=jax.ShapeDtypeStruct((b, p_total, f), jnp.float32),
    )(patch_features, queue)


# speedup vs baseline: 2.5258x; 1.1012x over previous
import functools

import jax
import jax.numpy as jnp
from jax.experimental import pallas as pl


def _body(a_ref, q_ref, o_ref, *, pblk):
    m = q_ref.shape[1]
    iota = jax.lax.broadcasted_iota(jnp.int32, (1, m), 1)
    sims = [jax.lax.dot_general(
        a_ref[:, p, :], q_ref[p], (((1,), (1,)), ((), ())),
        preferred_element_type=jnp.float32) for p in range(pblk)]
    onehots = [(jnp.argmax(sim, axis=1)[:, None] == iota).astype(jnp.float32)
               for sim in sims]
    for p in range(pblk):
        o_ref[:, p, :] = jax.lax.dot_general(
            onehots[p], q_ref[p], (((1,), (0,)), ((), ())),
            preferred_element_type=jnp.float32)


@jax.jit
def kernel(patch_features, queue):
    b, p_total, f = patch_features.shape
    _, m, _ = queue.shape
    pblk = 16
    grid = (p_total // pblk,)
    return pl.pallas_call(
        functools.partial(_body, pblk=pblk),
        grid=grid,
        in_specs=[
            pl.BlockSpec((b, pblk, f), lambda i: (0, i, 0)),
            pl.BlockSpec((pblk, m, f), lambda i: (i, 0, 0)),
        ],
        out_specs=pl.BlockSpec((b, pblk, f), lambda i: (0, i, 0)),
        out_shape=jax.ShapeDtypeStruct((b, p_total, f), jnp.float32),
    )(patch_features, queue)


# speedup vs baseline: 2.5861x; 1.0239x over previous
import functools

import jax
import jax.numpy as jnp
from jax.experimental import pallas as pl


def _body(a_ref, q_ref, o_ref, *, pblk):
    m = q_ref.shape[1]
    iota = jax.lax.broadcasted_iota(jnp.int32, (1, m), 1)
    sims = [jax.lax.dot_general(
        a_ref[:, p, :], q_ref[p], (((1,), (1,)), ((), ())),
        preferred_element_type=jnp.float32) for p in range(pblk)]
    onehots = [(jnp.argmax(sim, axis=1)[:, None] == iota).astype(jnp.float32)
               for sim in sims]
    for p in range(pblk):
        o_ref[:, p, :] = jax.lax.dot_general(
            onehots[p], q_ref[p], (((1,), (0,)), ((), ())),
            preferred_element_type=jnp.float32)


@jax.jit
def kernel(patch_features, queue):
    b, p_total, f = patch_features.shape
    _, m, _ = queue.shape
    pblk = 32
    grid = (p_total // pblk,)
    return pl.pallas_call(
        functools.partial(_body, pblk=pblk),
        grid=grid,
        in_specs=[
            pl.BlockSpec((b, pblk, f), lambda i: (0, i, 0)),
            pl.BlockSpec((pblk, m, f), lambda i: (i, 0, 0)),
        ],
        out_specs=pl.BlockSpec((b, pblk, f), lambda i: (0, i, 0)),
        out_shape=jax.ShapeDtypeStruct((b, p_total, f), jnp.float32),
    )(patch_features, queue)


# speedup vs baseline: 2.6008x; 1.0057x over previous
import functools

import jax
import jax.numpy as jnp
from jax.experimental import pallas as pl
from jax.experimental.pallas import tpu as pltpu


def _body(a_ref, q_ref, o_ref, *, pblk):
    m = q_ref.shape[1]
    iota = jax.lax.broadcasted_iota(jnp.int32, (1, m), 1)
    sims = [jax.lax.dot_general(
        a_ref[:, p, :], q_ref[p], (((1,), (1,)), ((), ())),
        preferred_element_type=jnp.float32) for p in range(pblk)]
    onehots = [(jnp.argmax(sim, axis=1)[:, None] == iota).astype(jnp.float32)
               for sim in sims]
    for p in range(pblk):
        o_ref[:, p, :] = jax.lax.dot_general(
            onehots[p], q_ref[p], (((1,), (0,)), ((), ())),
            preferred_element_type=jnp.float32)


@jax.jit
def kernel(patch_features, queue):
    b, p_total, f = patch_features.shape
    _, m, _ = queue.shape
    pblk = 32
    grid = (p_total // pblk,)
    return pl.pallas_call(
        functools.partial(_body, pblk=pblk),
        grid=grid,
        in_specs=[
            pl.BlockSpec((b, pblk, f), lambda i: (0, i, 0)),
            pl.BlockSpec((pblk, m, f), lambda i: (i, 0, 0)),
        ],
        out_specs=pl.BlockSpec((b, pblk, f), lambda i: (0, i, 0)),
        out_shape=jax.ShapeDtypeStruct((b, p_total, f), jnp.float32),
        compiler_params=pltpu.CompilerParams(
            dimension_semantics=("parallel",)),
    )(patch_features, queue)
